# Initial kernel scaffold; baseline (speedup 1.0000x reference)
import jax
import jax.numpy as jnp
from jax.experimental import pallas as pl


def kernel(x, batch, W_ih, W_hh, b_ih, b_hh):
    raise NotImplementedError("write your pallas kernel here")


# speedup vs baseline: 9.4391x; 9.4391x over previous
import functools

import jax
import jax.numpy as jnp
import numpy as np
from jax import lax
from jax.experimental import pallas as pl
from jax.experimental.pallas import tpu as pltpu
from jax.experimental.pallas import tpu_sc as plsc

NG = 512
DIM = 256
NCHUNK = DIM // 16
TILE_R = 128
NWORK = 32
SEG_PER_W = NG // NWORK
NEG = -1e30


def _iota():
    return lax.iota(jnp.int32, 16)


def _vfull(val):
    return jnp.broadcast_to(jnp.float32(val), (16,))


def _perm(v, k):
    return v.at[_iota() ^ k].get(mode="promise_in_bounds")


def _splat(v, j):
    return v.at[_iota() * 0 + j].get(mode="promise_in_bounds")


def _bcast_sum(v):
    for k in (1, 2, 4, 8):
        v = v + _perm(v, k)
    return v


def _bcast_max(v):
    for k in (1, 2, 4, 8):
        v = jnp.maximum(v, _perm(v, k))
    return v


def _sc_pool_body(x_hbm, segoff_hbm, q_hbm, out_hbm, x_tile, q_row, out_stage,
                  off_a, off_b):
    cid = lax.axis_index("c")
    sid = lax.axis_index("s")
    w = sid * 2 + cid
    seg0 = w * SEG_PER_W
    lanes = lax.iota(jnp.int32, 16)

    pltpu.sync_copy(segoff_hbm.at[pl.ds(seg0, 16)], off_a)
    pltpu.sync_copy(segoff_hbm.at[pl.ds(seg0 + 8, 16)], off_b)
    off_a_v = off_a[...]
    off_b_v = off_b[...]
    offs = [off_a_v[k] for k in range(16)] + [off_b_v[8]]

    def _sel(i):
        acc = offs[0]
        for k in range(1, 17):
            acc = jnp.where(i == k, offs[k], acc)
        return acc

    def seg_body(gi, _):
        o0 = _sel(gi)
        o1 = _sel(gi + 1)
        nrows = o1 - o0
        pltpu.sync_copy(q_hbm.at[pl.ds((seg0 + gi) * DIM, DIM)], q_row)

        for k in range(NCHUNK):
            out_stage[pl.ds(gi * DIM + k * 16, 16)] = _vfull(0.0)

        ntiles = (nrows + (TILE_R - 1)) // TILE_R

        def tile_body(t, carry):
            m_vec, l_vec = carry
            base = o0 + t * TILE_R
            pltpu.sync_copy(x_hbm.at[pl.ds(base * DIM, TILE_R * DIM)], x_tile)
            rcount = jnp.minimum(TILE_R, nrows - t * TILE_R)
            ngroups = (rcount + 15) // 16

            def group_body(g, carry2):
                m_v, l_v = carry2
                rb = g * 16

                def row_a(j, s_vec):
                    rowoff = (rb + j) * DIM
                    pacc = _vfull(0.0)
                    for k in range(NCHUNK):
                        pacc = pacc + (x_tile[pl.ds(rowoff + k * 16, 16)] *
                                       q_row[pl.ds(k * 16, 16)])
                    s = _bcast_sum(pacc)
                    return jnp.where(lanes == j, s, s_vec)

                s_vec = lax.fori_loop(0, 16, row_a,
                                      _vfull(NEG))
                row_in_seg = t * TILE_R + rb + lanes
                s_vec = jnp.where(row_in_seg < nrows, s_vec,
                                  _vfull(NEG))

                m_new = jnp.maximum(m_v, _bcast_max(s_vec))
                alpha = jnp.exp(m_v - m_new)
                p_vec = jnp.exp(s_vec - m_new)
                l_new = l_v * alpha + _bcast_sum(p_vec)

                p_s = [_splat(p_vec, jj) for jj in range(16)]

                def chunk_b(k, _c):
                    col = k * 16
                    a = out_stage[pl.ds(gi * DIM + col, 16)] * alpha
                    for jj in range(16):
                        a = a + p_s[jj] * x_tile[pl.ds((rb + jj) * DIM + col,
                                                       16)]
                    out_stage[pl.ds(gi * DIM + col, 16)] = a
                    return _c

                lax.fori_loop(0, NCHUNK, chunk_b, 0)
                return m_new, l_new

            return lax.fori_loop(0, ngroups, group_body, (m_vec, l_vec))

        init = (_vfull(0.0), _vfull(0.0))
        m_vec, l_vec = lax.fori_loop(0, ntiles, tile_body, init)

        inv = 1.0 / (l_vec + 1e-8)
        for k in range(NCHUNK):
            out_stage[pl.ds(gi * DIM + k * 16, 16)] = (
                out_stage[pl.ds(gi * DIM + k * 16, 16)] * inv)
        return 0

    lax.fori_loop(0, SEG_PER_W, seg_body, 0)
    pltpu.sync_copy(out_stage, out_hbm.at[pl.ds(seg0 * DIM, SEG_PER_W * DIM)])


def _make_sc_pool():
    mesh = plsc.VectorSubcoreMesh(core_axis_name="c", subcore_axis_name="s")
    return functools.partial(
        pl.kernel,
        mesh=mesh,
        out_type=jax.ShapeDtypeStruct((NG * DIM,), jnp.float32),
        scratch_types=[
            pltpu.VMEM((TILE_R * DIM,), jnp.float32),
            pltpu.VMEM((DIM,), jnp.float32),
            pltpu.VMEM((SEG_PER_W * DIM,), jnp.float32),
            pltpu.VMEM((16,), jnp.int32),
            pltpu.VMEM((16,), jnp.int32),
        ],
    )(_sc_pool_body)


def _lstm_tc(r_ref, h_ref, c_ref, wih_ref, whh_ref, b_ref, h_out, c_out):
    r = r_ref[...]
    h = h_ref[...]
    gates = (lax.dot_general(r, wih_ref[...], (((1,), (1,)), ((), ())),
                             preferred_element_type=jnp.float32)
             + lax.dot_general(h, whh_ref[...], (((1,), (1,)), ((), ())),
                               preferred_element_type=jnp.float32)
             + b_ref[...][None, :])
    i = jax.nn.sigmoid(gates[:, :DIM])
    f = jax.nn.sigmoid(gates[:, DIM:2 * DIM])
    g = jnp.tanh(gates[:, 2 * DIM:3 * DIM])
    o = jax.nn.sigmoid(gates[:, 3 * DIM:])
    c_new = f * c_ref[...] + i * g
    h_out[...] = o * jnp.tanh(c_new)
    c_out[...] = c_new


def _lstm_call(r, h, c, W_ih, W_hh, b):
    return pl.pallas_call(
        _lstm_tc,
        out_shape=[jax.ShapeDtypeStruct((NG, DIM), jnp.float32),
                   jax.ShapeDtypeStruct((NG, DIM), jnp.float32)],
    )(r, h, c, W_ih, W_hh, b)


def kernel(x, batch, W_ih, W_hh, b_ih, b_hh):
    n = x.shape[0]
    npad = n + TILE_R + 48
    xp = jnp.pad(x, ((0, npad - n), (0, 0))).reshape(-1)
    seg_off = jnp.searchsorted(batch, jnp.arange(NG + 1, dtype=jnp.int32),
                               side="left").astype(jnp.int32)
    seg_off_p = jnp.pad(seg_off, (0, 527 - NG))
    b = (b_ih + b_hh).astype(jnp.float32)

    sc_pool = _make_sc_pool()
    h = jnp.zeros((NG, DIM), jnp.float32)
    c = jnp.zeros((NG, DIM), jnp.float32)
    readout = None
    for _ in range(6):
        readout = sc_pool(xp, seg_off_p, h.reshape(-1)).reshape(NG, DIM)
        h, c = _lstm_call(readout, h, c, W_ih, W_hh, b)
    return jnp.concatenate([h, readout], axis=-1)


# speedup vs baseline: 11.8261x; 1.2529x over previous
import functools

import jax
import jax.numpy as jnp
import numpy as np
from jax import lax
from jax.experimental import pallas as pl
from jax.experimental.pallas import tpu as pltpu
from jax.experimental.pallas import tpu_sc as plsc

NG = 512
DIM = 256
NCHUNK = DIM // 16
TILE_R = 128
NWORK = 32
SEG_PER_W = NG // NWORK
NEG = -1e30


def _iota():
    return lax.iota(jnp.int32, 16)


def _vfull(val):
    return jnp.broadcast_to(jnp.float32(val), (16,))


def _perm(v, k):
    return v.at[_iota() ^ k].get(mode="promise_in_bounds")


def _splat(v, j):
    return v.at[_iota() * 0 + j].get(mode="promise_in_bounds")


def _bcast_sum(v):
    for k in (1, 2, 4, 8):
        v = v + _perm(v, k)
    return v


def _bcast_max(v):
    for k in (1, 2, 4, 8):
        v = jnp.maximum(v, _perm(v, k))
    return v


def _sc_pool_body(x_hbm, segoff_hbm, q_hbm, out_hbm, x_tile, x_tile_b, q_row,
                  q_row_b, out_stage, off_a, off_b, sem_xa, sem_xb, sem_qa,
                  sem_qb):
    cid = lax.axis_index("c")
    sid = lax.axis_index("s")
    w = sid * 2 + cid
    seg0 = w * SEG_PER_W
    lanes = lax.iota(jnp.int32, 16)

    pltpu.sync_copy(segoff_hbm.at[pl.ds(seg0, 16)], off_a)
    pltpu.sync_copy(segoff_hbm.at[pl.ds(seg0 + 8, 16)], off_b)
    off_a_v = off_a[...]
    off_b_v = off_b[...]
    offs = [off_a_v[k] for k in range(16)] + [off_b_v[8]]

    def _sel(i):
        acc = offs[0]
        for k in range(1, 17):
            acc = jnp.where(i == k, offs[k], acc)
        return acc

    def process_tile(buf, q_buf, gi, t, nrows, m_v0, l_v0):
        rcount = jnp.minimum(TILE_R, nrows - t * TILE_R)
        ngroups = (rcount + 15) // 16

        def group_body(g, carry2):
            m_v, l_v = carry2
            rb = g * 16

            def row_a(j, s_vec):
                rowoff = (rb + j) * DIM
                pacc = _vfull(0.0)
                for k in range(NCHUNK):
                    pacc = pacc + (buf[pl.ds(rowoff + k * 16, 16)] *
                                   q_buf[pl.ds(k * 16, 16)])
                s = _bcast_sum(pacc)
                return jnp.where(lanes == j, s, s_vec)

            s_vec = lax.fori_loop(0, 16, row_a, _vfull(NEG))
            row_in_seg = t * TILE_R + rb + lanes
            s_vec = jnp.where(row_in_seg < nrows, s_vec, _vfull(NEG))

            m_new = jnp.maximum(m_v, _bcast_max(s_vec))
            alpha = jnp.exp(m_v - m_new)
            p_vec = jnp.exp(s_vec - m_new)
            l_new = l_v * alpha + _bcast_sum(p_vec)

            p_s = [_splat(p_vec, jj) for jj in range(16)]
            for k in range(NCHUNK):
                col = k * 16
                a = out_stage[pl.ds(gi * DIM + col, 16)] * alpha
                for jj in range(16):
                    a = a + p_s[jj] * buf[pl.ds((rb + jj) * DIM + col, 16)]
                out_stage[pl.ds(gi * DIM + col, 16)] = a
            return m_new, l_new

        return lax.fori_loop(0, ngroups, group_body, (m_v0, l_v0))

    def seg_work(gi, mybuf, myq, sem_x, sem_q, nxtbuf, nxtq, nsem_x, nsem_q):
        o0 = _sel(gi)
        o1 = _sel(gi + 1)
        nrows = o1 - o0
        gin = gi + 1

        @pl.when(gin < SEG_PER_W)
        def _():
            o0n = _sel(gin)
            pltpu.async_copy(x_hbm.at[pl.ds(o0n * DIM, TILE_R * DIM)],
                             nxtbuf, nsem_x)
            pltpu.async_copy(q_hbm.at[pl.ds((seg0 + gin) * DIM, DIM)],
                             nxtq, nsem_q)

        pltpu.make_async_copy(x_hbm.at[pl.ds(o0 * DIM, TILE_R * DIM)],
                              mybuf, sem_x).wait()
        pltpu.make_async_copy(q_hbm.at[pl.ds((seg0 + gi) * DIM, DIM)],
                              myq, sem_q).wait()

        for k in range(NCHUNK):
            out_stage[pl.ds(gi * DIM + k * 16, 16)] = _vfull(0.0)

        ntiles = (nrows + (TILE_R - 1)) // TILE_R
        m_vec, l_vec = process_tile(mybuf, myq, gi, 0, nrows,
                                    _vfull(0.0), _vfull(0.0))

        def tile_body(t, carry):
            m_v, l_v = carry
            base = o0 + t * TILE_R
            pltpu.sync_copy(x_hbm.at[pl.ds(base * DIM, TILE_R * DIM)], mybuf)
            return process_tile(mybuf, myq, gi, t, nrows, m_v, l_v)

        m_vec, l_vec = lax.fori_loop(1, ntiles, tile_body, (m_vec, l_vec))

        inv = 1.0 / (l_vec + 1e-8)
        for k in range(NCHUNK):
            out_stage[pl.ds(gi * DIM + k * 16, 16)] = (
                out_stage[pl.ds(gi * DIM + k * 16, 16)] * inv)

    pltpu.async_copy(x_hbm.at[pl.ds(offs[0] * DIM, TILE_R * DIM)],
                     x_tile, sem_xa)
    pltpu.async_copy(q_hbm.at[pl.ds(seg0 * DIM, DIM)], q_row, sem_qa)

    def pair_body(p, _):
        gi = 2 * p
        seg_work(gi, x_tile, q_row, sem_xa, sem_qa,
                 x_tile_b, q_row_b, sem_xb, sem_qb)
        seg_work(gi + 1, x_tile_b, q_row_b, sem_xb, sem_qb,
                 x_tile, q_row, sem_xa, sem_qa)
        return 0

    lax.fori_loop(0, SEG_PER_W // 2, pair_body, 0)
    pltpu.sync_copy(out_stage, out_hbm.at[pl.ds(seg0 * DIM, SEG_PER_W * DIM)])


def _make_sc_pool():
    mesh = plsc.VectorSubcoreMesh(core_axis_name="c", subcore_axis_name="s")
    return functools.partial(
        pl.kernel,
        mesh=mesh,
        out_type=jax.ShapeDtypeStruct((NG * DIM,), jnp.float32),
        scratch_types=[
            pltpu.VMEM((TILE_R * DIM,), jnp.float32),
            pltpu.VMEM((TILE_R * DIM,), jnp.float32),
            pltpu.VMEM((DIM,), jnp.float32),
            pltpu.VMEM((DIM,), jnp.float32),
            pltpu.VMEM((SEG_PER_W * DIM,), jnp.float32),
            pltpu.VMEM((16,), jnp.int32),
            pltpu.VMEM((16,), jnp.int32),
            pltpu.SemaphoreType.DMA,
            pltpu.SemaphoreType.DMA,
            pltpu.SemaphoreType.DMA,
            pltpu.SemaphoreType.DMA,
        ],
    )(_sc_pool_body)


def _lstm_tc(r_ref, h_ref, c_ref, wih_ref, whh_ref, b_ref, h_out, c_out):
    r = r_ref[...]
    h = h_ref[...]
    gates = (lax.dot_general(r, wih_ref[...], (((1,), (1,)), ((), ())),
                             preferred_element_type=jnp.float32)
             + lax.dot_general(h, whh_ref[...], (((1,), (1,)), ((), ())),
                               preferred_element_type=jnp.float32)
             + b_ref[...][None, :])
    i = jax.nn.sigmoid(gates[:, :DIM])
    f = jax.nn.sigmoid(gates[:, DIM:2 * DIM])
    g = jnp.tanh(gates[:, 2 * DIM:3 * DIM])
    o = jax.nn.sigmoid(gates[:, 3 * DIM:])
    c_new = f * c_ref[...] + i * g
    h_out[...] = o * jnp.tanh(c_new)
    c_out[...] = c_new


def _lstm_call(r, h, c, W_ih, W_hh, b):
    return pl.pallas_call(
        _lstm_tc,
        out_shape=[jax.ShapeDtypeStruct((NG, DIM), jnp.float32),
                   jax.ShapeDtypeStruct((NG, DIM), jnp.float32)],
    )(r, h, c, W_ih, W_hh, b)


def kernel(x, batch, W_ih, W_hh, b_ih, b_hh):
    n = x.shape[0]
    npad = n + TILE_R + 48
    xp = jnp.pad(x, ((0, npad - n), (0, 0))).reshape(-1)
    seg_off = jnp.searchsorted(batch, jnp.arange(NG + 1, dtype=jnp.int32),
                               side="left").astype(jnp.int32)
    seg_off_p = jnp.pad(seg_off, (0, 527 - NG))
    b = (b_ih + b_hh).astype(jnp.float32)

    sc_pool = _make_sc_pool()
    h = jnp.zeros((NG, DIM), jnp.float32)
    c = jnp.zeros((NG, DIM), jnp.float32)
    readout = None
    for _ in range(6):
        readout = sc_pool(xp, seg_off_p, h.reshape(-1)).reshape(NG, DIM)
        h, c = _lstm_call(readout, h, c, W_ih, W_hh, b)
    return jnp.concatenate([h, readout], axis=-1)


# speedup vs baseline: 14.1194x; 1.1939x over previous
import functools

import jax
import jax.numpy as jnp
import numpy as np
from jax import lax
from jax.experimental import pallas as pl
from jax.experimental.pallas import tpu as pltpu
from jax.experimental.pallas import tpu_sc as plsc

NG = 512
DIM = 256
NCHUNK = DIM // 16
TILE_R = 128
NWORK = 32
SEG_PER_W = NG // NWORK
NEG = -1e30


def _iota():
    return lax.iota(jnp.int32, 16)


def _vfull(val):
    return jnp.broadcast_to(jnp.float32(val), (16,))


def _perm(v, k):
    return v.at[_iota() ^ k].get(mode="promise_in_bounds")


def _splat(v, j):
    return v.at[_iota() * 0 + j].get(mode="promise_in_bounds")


def _bcast_sum(v):
    for k in (1, 2, 4, 8):
        v = v + _perm(v, k)
    return v


def _bcast_max(v):
    for k in (1, 2, 4, 8):
        v = jnp.maximum(v, _perm(v, k))
    return v


def _sc_pool_body(x_hbm, segoff_hbm, q_hbm, out_hbm, x_tile, x_tile_b, q_row,
                  q_row_b, out_stage, off_a, off_b, sem_xa, sem_xb, sem_qa,
                  sem_qb):
    cid = lax.axis_index("c")
    sid = lax.axis_index("s")
    w = sid * 2 + cid
    seg0 = w * SEG_PER_W
    lanes = lax.iota(jnp.int32, 16)

    pltpu.sync_copy(segoff_hbm.at[pl.ds(seg0, 16)], off_a)
    pltpu.sync_copy(segoff_hbm.at[pl.ds(seg0 + 8, 16)], off_b)
    off_a_v = off_a[...]
    off_b_v = off_b[...]
    offs = [off_a_v[k] for k in range(16)] + [off_b_v[8]]

    def _sel(i):
        acc = offs[0]
        for k in range(1, 17):
            acc = jnp.where(i == k, offs[k], acc)
        return acc

    def process_tile(buf, q_buf, gi, t, nrows, m_v0, l_v0):
        rcount = jnp.minimum(TILE_R, nrows - t * TILE_R)
        ngroups = (rcount + 15) // 16

        def group_body(g, carry2):
            m_v, l_v = carry2
            rb = g * 16

            def row_a(j, s_vec):
                rowoff = (rb + j) * DIM
                pacc = _vfull(0.0)
                for k in range(NCHUNK):
                    pacc = pacc + (buf[pl.ds(rowoff + k * 16, 16)] *
                                   q_buf[pl.ds(k * 16, 16)])
                s = _bcast_sum(pacc)
                return jnp.where(lanes == j, s, s_vec)

            s_vec = lax.fori_loop(0, 16, row_a, _vfull(NEG))
            row_in_seg = t * TILE_R + rb + lanes
            s_vec = jnp.where(row_in_seg < nrows, s_vec, _vfull(NEG))

            m_new = jnp.maximum(m_v, _bcast_max(s_vec))
            alpha = jnp.exp(m_v - m_new)
            p_vec = jnp.exp(s_vec - m_new)
            l_new = l_v * alpha + _bcast_sum(p_vec)

            p_s = [_splat(p_vec, jj) for jj in range(16)]
            for k in range(NCHUNK):
                col = k * 16
                acc4 = [out_stage[pl.ds(gi * DIM + col, 16)] * alpha,
                        _vfull(0.0), _vfull(0.0), _vfull(0.0)]
                for jj in range(16):
                    acc4[jj % 4] = (acc4[jj % 4] +
                                    p_s[jj] * buf[pl.ds((rb + jj) * DIM + col,
                                                        16)])
                out_stage[pl.ds(gi * DIM + col, 16)] = (
                    (acc4[0] + acc4[1]) + (acc4[2] + acc4[3]))
            return m_new, l_new

        return lax.fori_loop(0, ngroups, group_body, (m_v0, l_v0))

    def seg_work(gi, mybuf, myq, sem_x, sem_q, nxtbuf, nxtq, nsem_x, nsem_q):
        o0 = _sel(gi)
        o1 = _sel(gi + 1)
        nrows = o1 - o0
        gin = gi + 1

        @pl.when(gin < SEG_PER_W)
        def _():
            o0n = _sel(gin)
            pltpu.async_copy(x_hbm.at[pl.ds(o0n * DIM, TILE_R * DIM)],
                             nxtbuf, nsem_x)
            pltpu.async_copy(q_hbm.at[pl.ds((seg0 + gin) * DIM, DIM)],
                             nxtq, nsem_q)

        pltpu.make_async_copy(x_hbm.at[pl.ds(o0 * DIM, TILE_R * DIM)],
                              mybuf, sem_x).wait()
        pltpu.make_async_copy(q_hbm.at[pl.ds((seg0 + gi) * DIM, DIM)],
                              myq, sem_q).wait()

        for k in range(NCHUNK):
            out_stage[pl.ds(gi * DIM + k * 16, 16)] = _vfull(0.0)

        ntiles = (nrows + (TILE_R - 1)) // TILE_R
        m_vec, l_vec = process_tile(mybuf, myq, gi, 0, nrows,
                                    _vfull(0.0), _vfull(0.0))

        def tile_body(t, carry):
            m_v, l_v = carry
            base = o0 + t * TILE_R
            pltpu.sync_copy(x_hbm.at[pl.ds(base * DIM, TILE_R * DIM)], mybuf)
            return process_tile(mybuf, myq, gi, t, nrows, m_v, l_v)

        m_vec, l_vec = lax.fori_loop(1, ntiles, tile_body, (m_vec, l_vec))

        inv = 1.0 / (l_vec + 1e-8)
        for k in range(NCHUNK):
            out_stage[pl.ds(gi * DIM + k * 16, 16)] = (
                out_stage[pl.ds(gi * DIM + k * 16, 16)] * inv)

    pltpu.async_copy(x_hbm.at[pl.ds(offs[0] * DIM, TILE_R * DIM)],
                     x_tile, sem_xa)
    pltpu.async_copy(q_hbm.at[pl.ds(seg0 * DIM, DIM)], q_row, sem_qa)

    def pair_body(p, _):
        gi = 2 * p
        seg_work(gi, x_tile, q_row, sem_xa, sem_qa,
                 x_tile_b, q_row_b, sem_xb, sem_qb)
        seg_work(gi + 1, x_tile_b, q_row_b, sem_xb, sem_qb,
                 x_tile, q_row, sem_xa, sem_qa)
        return 0

    lax.fori_loop(0, SEG_PER_W // 2, pair_body, 0)
    pltpu.sync_copy(out_stage, out_hbm.at[pl.ds(seg0 * DIM, SEG_PER_W * DIM)])


def _make_sc_pool():
    mesh = plsc.VectorSubcoreMesh(core_axis_name="c", subcore_axis_name="s")
    return functools.partial(
        pl.kernel,
        mesh=mesh,
        out_type=jax.ShapeDtypeStruct((NG * DIM,), jnp.float32),
        scratch_types=[
            pltpu.VMEM((TILE_R * DIM,), jnp.float32),
            pltpu.VMEM((TILE_R * DIM,), jnp.float32),
            pltpu.VMEM((DIM,), jnp.float32),
            pltpu.VMEM((DIM,), jnp.float32),
            pltpu.VMEM((SEG_PER_W * DIM,), jnp.float32),
            pltpu.VMEM((16,), jnp.int32),
            pltpu.VMEM((16,), jnp.int32),
            pltpu.SemaphoreType.DMA,
            pltpu.SemaphoreType.DMA,
            pltpu.SemaphoreType.DMA,
            pltpu.SemaphoreType.DMA,
        ],
    )(_sc_pool_body)


def _lstm_tc(r_ref, h_ref, c_ref, wih_ref, whh_ref, b_ref, h_out, c_out):
    r = r_ref[...]
    h = h_ref[...]
    gates = (lax.dot_general(r, wih_ref[...], (((1,), (1,)), ((), ())),
                             preferred_element_type=jnp.float32)
             + lax.dot_general(h, whh_ref[...], (((1,), (1,)), ((), ())),
                               preferred_element_type=jnp.float32)
             + b_ref[...][None, :])
    i = jax.nn.sigmoid(gates[:, :DIM])
    f = jax.nn.sigmoid(gates[:, DIM:2 * DIM])
    g = jnp.tanh(gates[:, 2 * DIM:3 * DIM])
    o = jax.nn.sigmoid(gates[:, 3 * DIM:])
    c_new = f * c_ref[...] + i * g
    h_out[...] = o * jnp.tanh(c_new)
    c_out[...] = c_new


def _lstm_call(r, h, c, W_ih, W_hh, b):
    return pl.pallas_call(
        _lstm_tc,
        out_shape=[jax.ShapeDtypeStruct((NG, DIM), jnp.float32),
                   jax.ShapeDtypeStruct((NG, DIM), jnp.float32)],
    )(r, h, c, W_ih, W_hh, b)


def kernel(x, batch, W_ih, W_hh, b_ih, b_hh):
    n = x.shape[0]
    npad = n + TILE_R + 48
    xp = jnp.pad(x, ((0, npad - n), (0, 0))).reshape(-1)
    seg_off = jnp.searchsorted(batch, jnp.arange(NG + 1, dtype=jnp.int32),
                               side="left").astype(jnp.int32)
    seg_off_p = jnp.pad(seg_off, (0, 527 - NG))
    b = (b_ih + b_hh).astype(jnp.float32)

    sc_pool = _make_sc_pool()
    h = jnp.zeros((NG, DIM), jnp.float32)
    c = jnp.zeros((NG, DIM), jnp.float32)
    readout = None
    for _ in range(6):
        readout = sc_pool(xp, seg_off_p, h.reshape(-1)).reshape(NG, DIM)
        h, c = _lstm_call(readout, h, c, W_ih, W_hh, b)
    return jnp.concatenate([h, readout], axis=-1)
